# baseline (speedup 1.0000x reference)
import functools

import jax
import jax.numpy as jnp
from jax import lax
from jax.experimental import pallas as pl
from jax.experimental.pallas import tpu as pltpu
from jax.experimental.pallas import tpu_sc as plsc

N_VOXELS = 100000
N_POINTS = 320000
D = 128
B = 16

NC = 2
NS = 16
L = 16
NW = NC * NS

P_TILE = N_POINTS // NW
CH = 128
CHK = 1280
NCHK = -(-P_TILE // CHK)
TAIL = P_TILE - (NCHK - 1) * CHK

C_BINS = B * N_VOXELS
DUMP_B = B
ZCH = 10240
Z_SLICE = 10 * ZCH
C_TOTAL = NS * Z_SLICE
F_SLICE = C_BINS // NS
FCH = 5120
NFCH = -(-F_SLICE // FCH)
F_TAIL = F_SLICE - (NFCH - 1) * FCH


def _hist_body(v2p_hbm, bat_hbm, out_hbm, vb, bb, fidx, ones, zfb, cs,
               sem_a, sem_f0, sem_f1):
    c = lax.axis_index("c")
    s = lax.axis_index("s")
    wid = c * NS + s
    base = wid * P_TILE
    hsems = (sem_f0, sem_f1)

    def zbody(i, carry):
        zfb[pl.ds(i * L, L)] = jnp.zeros((L,), jnp.float32)
        return carry

    lax.fori_loop(0, ZCH // L, zbody, 0)
    zero_descs = [
        pltpu.async_copy(zfb, cs.at[pl.ds(s * Z_SLICE + k * ZCH, ZCH)], sem_a)
        for k in range(Z_SLICE // ZCH)
    ]

    def obody(j, carry):
        ones[pl.ds(j * L, L)] = jnp.ones((L,), jnp.float32)
        return carry

    lax.fori_loop(0, CHK // L, obody, 0)

    def _fire_loads(t):
        h = t % 2
        n = CHK if t < NCHK - 1 else TAIL
        off = base + t * CHK
        return [
            pltpu.async_copy(
                v2p_hbm.at[pl.ds(off, n)], vb.at[pl.ds(h * CHK, n)], hsems[h]
            ),
            pltpu.async_copy(
                bat_hbm.at[pl.ds(off, n)], bb.at[pl.ds(h * CHK, n)], hsems[h]
            ),
        ]

    pending = _fire_loads(0)
    for t in range(NCHK):
        nxt = _fire_loads(t + 1) if t + 1 < NCHK else None
        for d in pending:
            d.wait()
        h = t % 2
        if t == NCHK - 1:
            for u in range((CHK - TAIL) // L):
                vb[pl.ds(h * CHK + TAIL + u * L, L)] = jnp.zeros((L,), jnp.int32)
                bb[pl.ds(h * CHK + TAIL + u * L, L)] = jnp.full((L,), DUMP_B, jnp.int32)

        def fbody(j, carry):
            o = j * L
            v = vb[pl.ds(h * CHK + o, L)]
            b = bb[pl.ds(h * CHK + o, L)]
            fidx[pl.ds(t * CHK + o, L)] = b * N_VOXELS + v
            return carry

        lax.fori_loop(0, CHK // L, fbody, 0)
        pending = nxt

    for d in zero_descs:
        d.wait()
    plsc.subcore_barrier()

    scat_descs = [
        pltpu.async_copy(ones, cs.at[fidx.at[pl.ds(t * CHK, CHK)]], sem_a, add=True)
        for t in range(NCHK)
    ]
    for d in scat_descs:
        d.wait()

    plsc.subcore_barrier()

    def _n(k):
        return FCH if k < NFCH - 1 else F_TAIL

    hop1 = [None] * NFCH
    hop2 = [None] * NFCH
    for k in range(NFCH + 1):
        if k < NFCH:
            hh = k % 2
            if k >= 2:
                hop2[k - 2].wait()
            hop1[k] = pltpu.async_copy(
                cs.at[pl.ds(s * F_SLICE + k * FCH, _n(k))],
                zfb.at[pl.ds(hh * FCH, _n(k))],
                sem_a,
            )
        if k >= 1:
            hop1[k - 1].wait()
            hh = (k - 1) % 2
            hop2[k - 1] = pltpu.async_copy(
                zfb.at[pl.ds(hh * FCH, _n(k - 1))],
                out_hbm.at[pl.ds(c * C_BINS + s * F_SLICE + (k - 1) * FCH, _n(k - 1))],
                hsems[hh],
            )
    hop2[NFCH - 2].wait()
    hop2[NFCH - 1].wait()


def _histogram(v2p_map, batch_idxs):
    mesh = plsc.VectorSubcoreMesh(
        core_axis_name="c", subcore_axis_name="s", num_cores=NC, num_subcores=NS
    )
    return pl.kernel(
        _hist_body,
        out_type=jax.ShapeDtypeStruct((NC * C_BINS,), jnp.float32),
        mesh=mesh,
        scratch_types=[
            pltpu.VMEM((2 * CHK,), jnp.int32),
            pltpu.VMEM((2 * CHK,), jnp.int32),
            pltpu.VMEM((NCHK * CHK,), jnp.int32),
            pltpu.VMEM((CHK,), jnp.float32),
            pltpu.VMEM((2 * FCH,), jnp.float32),
            pltpu.VMEM_SHARED((C_TOTAL,), jnp.float32),
            pltpu.SemaphoreType.DMA,
            pltpu.SemaphoreType.DMA,
            pltpu.SemaphoreType.DMA,
        ],
    )(v2p_map, batch_idxs)


VSTEP = 16384
NKM = 98304 // VSTEP
V_TAIL = N_VOXELS - NKM * VSTEP
_DN_MK = (((1,), (0,)), ((), ()))
_PREC = lax.Precision.DEFAULT


def _mm_body(c_ref, a_ref, ct_ref, at_ref, cap_ref, ls_ref, out_ref, acc, cnt):
    i = pl.program_id(0)

    @pl.when(i == 0)
    def _init():
        acc[...] = jnp.zeros_like(acc)
        cnt[...] = jnp.zeros_like(cnt)

    @pl.when(i < NKM)
    def _step():
        c0 = c_ref[0]
        c1 = c_ref[1]
        ablk = a_ref[...]
        acc[...] += lax.dot_general(
            c0.astype(jnp.bfloat16), ablk, _DN_MK,
            preferred_element_type=jnp.float32, precision=_PREC
        ) + lax.dot_general(
            c1.astype(jnp.bfloat16), ablk, _DN_MK,
            preferred_element_type=jnp.float32, precision=_PREC
        )
        cnt[...] += jnp.sum(c0 + c1, axis=1, keepdims=True)

    @pl.when(i == NKM)
    def _tail():
        at = at_ref[...]
        sums = acc[...] + lax.dot_general(
            ct_ref[0].astype(jnp.bfloat16), at, _DN_MK,
            preferred_element_type=jnp.float32, precision=_PREC,
        ) + lax.dot_general(
            ct_ref[1].astype(jnp.bfloat16), at, _DN_MK,
            preferred_element_type=jnp.float32, precision=_PREC,
        )
        cnts = cnt[:, 0:1] + jnp.sum(
            ct_ref[0] + ct_ref[1], axis=1, keepdims=True
        )
        pooled = sums / jnp.maximum(cnts, 1.0)
        pn = pooled / jnp.maximum(
            jnp.sqrt(jnp.sum(pooled * pooled, axis=1, keepdims=True)), 1e-12
        )
        cap = cap_ref[...]
        cn = cap / jnp.maximum(
            jnp.sqrt(jnp.sum(cap * cap, axis=1, keepdims=True)), 1e-12
        )
        scale = jnp.exp(ls_ref[0, 0])
        out_ref[...] = (
            lax.dot_general(
                pn, cn, (((1,), (1,)), ((), ())),
                preferred_element_type=jnp.float32, precision=_PREC,
            )
            * scale
        )


def _pool_logits(counts_bm, adapter_feats, ct, at, caption_embed, ls2d):
    return pl.pallas_call(
        _mm_body,
        grid=(NKM + 1,),
        in_specs=[
            pl.BlockSpec((NC, B, VSTEP), lambda i: (0, 0, jnp.minimum(i, NKM - 1))),
            pl.BlockSpec((VSTEP, D), lambda i: (jnp.minimum(i, NKM - 1), 0)),
            pl.BlockSpec((NC, B, V_TAIL), lambda i: (0, 0, 0)),
            pl.BlockSpec((V_TAIL, D), lambda i: (0, 0)),
            pl.BlockSpec((B, D), lambda i: (0, 0)),
            pl.BlockSpec(memory_space=pltpu.SMEM),
        ],
        out_specs=pl.BlockSpec((B, B), lambda i: (0, 0)),
        out_shape=jax.ShapeDtypeStruct((B, B), jnp.float32),
        scratch_shapes=[
            pltpu.VMEM((B, D), jnp.float32),
            pltpu.VMEM((B, D), jnp.float32),
        ],
    )(counts_bm, adapter_feats, ct, at, caption_embed, ls2d)


def kernel(adapter_feats, v2p_map, batch_idxs, caption_embed, caption_idx, logit_scale):
    del caption_idx
    a_bf = adapter_feats.astype(jnp.bfloat16)
    counts_bm = _histogram(v2p_map, batch_idxs).reshape(NC, B, N_VOXELS)
    ct = lax.slice(counts_bm, (0, 0, NKM * VSTEP), (NC, B, N_VOXELS))
    at = lax.slice(a_bf, (NKM * VSTEP, 0), (N_VOXELS, D))
    ls2d = jnp.reshape(logit_scale, (1, 1))
    return _pool_logits(counts_bm, a_bf, ct, at, caption_embed, ls2d)


# speedup vs baseline: 1.0119x; 1.0119x over previous
import functools

import jax
import jax.numpy as jnp
from jax import lax
from jax.experimental import pallas as pl
from jax.experimental.pallas import tpu as pltpu
from jax.experimental.pallas import tpu_sc as plsc

N_VOXELS = 100000
N_POINTS = 320000
D = 128
B = 16

NC = 2
NS = 16
L = 16
NW = NC * NS

P_TILE = N_POINTS // NW
CH = 128
CHK = 1280
NCHK = -(-P_TILE // CHK)
TAIL = P_TILE - (NCHK - 1) * CHK

C_BINS = B * N_VOXELS
DUMP_B = B
ZCH = 10240
Z_SLICE = 10 * ZCH
C_TOTAL = NS * Z_SLICE
F_SLICE = C_BINS // NS
FCH = 5120
NFCH = -(-F_SLICE // FCH)
F_TAIL = F_SLICE - (NFCH - 1) * FCH


def _hist_body(v2p_hbm, bat_hbm, out_hbm, vb, bb, fidx, ones, zfb, cs,
               sem_a, sem_f0, sem_f1):
    c = lax.axis_index("c")
    s = lax.axis_index("s")
    wid = c * NS + s
    base = wid * P_TILE
    hsems = (sem_f0, sem_f1)

    def zbody(i, carry):
        zfb[pl.ds(i * L, L)] = jnp.zeros((L,), jnp.float32)
        return carry

    lax.fori_loop(0, ZCH // L, zbody, 0)
    zero_descs = [
        pltpu.async_copy(zfb, cs.at[pl.ds(s * Z_SLICE + k * ZCH, ZCH)], sem_a)
        for k in range(Z_SLICE // ZCH)
    ]

    def obody(j, carry):
        ones[pl.ds(j * L, L)] = jnp.ones((L,), jnp.float32)
        return carry

    lax.fori_loop(0, CHK // L, obody, 0)

    def _fire_loads(t):
        h = t % 2
        n = CHK if t < NCHK - 1 else TAIL
        off = base + t * CHK
        return [
            pltpu.async_copy(
                v2p_hbm.at[pl.ds(off, n)], vb.at[pl.ds(h * CHK, n)], hsems[h]
            ),
            pltpu.async_copy(
                bat_hbm.at[pl.ds(off, n)], bb.at[pl.ds(h * CHK, n)], hsems[h]
            ),
        ]

    pending = _fire_loads(0)
    for t in range(NCHK):
        nxt = _fire_loads(t + 1) if t + 1 < NCHK else None
        for d in pending:
            d.wait()
        h = t % 2
        if t == NCHK - 1:
            for u in range((CHK - TAIL) // L):
                vb[pl.ds(h * CHK + TAIL + u * L, L)] = jnp.zeros((L,), jnp.int32)
                bb[pl.ds(h * CHK + TAIL + u * L, L)] = jnp.full((L,), DUMP_B, jnp.int32)

        def fbody(j, carry):
            o = j * L
            v = vb[pl.ds(h * CHK + o, L)]
            b = bb[pl.ds(h * CHK + o, L)]
            fidx[pl.ds(t * CHK + o, L)] = b * N_VOXELS + v
            return carry

        lax.fori_loop(0, CHK // L, fbody, 0)
        pending = nxt

    for d in zero_descs:
        d.wait()
    plsc.subcore_barrier()

    scat_descs = [
        pltpu.async_copy(ones, cs.at[fidx.at[pl.ds(t * CHK, CHK)]], sem_a, add=True)
        for t in range(NCHK)
    ]
    for d in scat_descs:
        d.wait()

    plsc.subcore_barrier()

    def _n(k):
        return FCH if k < NFCH - 1 else F_TAIL

    hop1 = [None] * NFCH
    hop2 = [None] * NFCH
    for k in range(NFCH + 1):
        if k < NFCH:
            hh = k % 2
            if k >= 2:
                hop2[k - 2].wait()
            hop1[k] = pltpu.async_copy(
                cs.at[pl.ds(s * F_SLICE + k * FCH, _n(k))],
                zfb.at[pl.ds(hh * FCH, _n(k))],
                sem_a,
            )
        if k >= 1:
            hop1[k - 1].wait()
            hh = (k - 1) % 2
            hop2[k - 1] = pltpu.async_copy(
                zfb.at[pl.ds(hh * FCH, _n(k - 1))],
                out_hbm.at[pl.ds(c * C_BINS + s * F_SLICE + (k - 1) * FCH, _n(k - 1))],
                hsems[hh],
            )
    hop2[NFCH - 2].wait()
    hop2[NFCH - 1].wait()


def _histogram(v2p_map, batch_idxs):
    mesh = plsc.VectorSubcoreMesh(
        core_axis_name="c", subcore_axis_name="s", num_cores=NC, num_subcores=NS
    )
    return pl.kernel(
        _hist_body,
        out_type=jax.ShapeDtypeStruct((NC * C_BINS,), jnp.float32),
        mesh=mesh,
        scratch_types=[
            pltpu.VMEM((2 * CHK,), jnp.int32),
            pltpu.VMEM((2 * CHK,), jnp.int32),
            pltpu.VMEM((NCHK * CHK,), jnp.int32),
            pltpu.VMEM((CHK,), jnp.float32),
            pltpu.VMEM((2 * FCH,), jnp.float32),
            pltpu.VMEM_SHARED((C_TOTAL,), jnp.float32),
            pltpu.SemaphoreType.DMA,
            pltpu.SemaphoreType.DMA,
            pltpu.SemaphoreType.DMA,
        ],
    )(v2p_map, batch_idxs)


VSTEP = 16384
NKM = 98304 // VSTEP
V_TAIL = N_VOXELS - NKM * VSTEP
_DN_MK = (((1,), (0,)), ((), ()))
_PREC = lax.Precision.DEFAULT


def _mm_body(c_ref, a_ref, ct_ref, at_ref, cap_ref, ls_ref, out_ref, acc, cnt):
    i = pl.program_id(0)

    @pl.when(i == 0)
    def _init():
        acc[...] = jnp.zeros_like(acc)
        cnt[...] = jnp.zeros_like(cnt)

    @pl.when(i < NKM)
    def _step():
        c0 = c_ref[0]
        c1 = c_ref[1]
        ablk = a_ref[...]
        acc[...] += lax.dot_general(
            c0, ablk, _DN_MK, preferred_element_type=jnp.float32, precision=_PREC
        ) + lax.dot_general(
            c1, ablk, _DN_MK, preferred_element_type=jnp.float32, precision=_PREC
        )
        cnt[...] += jnp.sum(c0 + c1, axis=1, keepdims=True)

    @pl.when(i == NKM)
    def _tail():
        at = at_ref[...]
        sums = acc[...] + lax.dot_general(
            ct_ref[0], at, _DN_MK, preferred_element_type=jnp.float32,
            precision=_PREC,
        ) + lax.dot_general(
            ct_ref[1], at, _DN_MK, preferred_element_type=jnp.float32,
            precision=_PREC,
        )
        cnts = cnt[:, 0:1] + jnp.sum(
            ct_ref[0] + ct_ref[1], axis=1, keepdims=True
        )
        pooled = sums / jnp.maximum(cnts, 1.0)
        pn = pooled / jnp.maximum(
            jnp.sqrt(jnp.sum(pooled * pooled, axis=1, keepdims=True)), 1e-12
        )
        cap = cap_ref[...]
        cn = cap / jnp.maximum(
            jnp.sqrt(jnp.sum(cap * cap, axis=1, keepdims=True)), 1e-12
        )
        scale = jnp.exp(ls_ref[0, 0])
        out_ref[...] = (
            lax.dot_general(
                pn, cn, (((1,), (1,)), ((), ())),
                preferred_element_type=jnp.float32, precision=_PREC,
            )
            * scale
        )


def _pool_logits(counts_bm, adapter_feats, ct, at, caption_embed, ls2d):
    return pl.pallas_call(
        _mm_body,
        grid=(NKM + 1,),
        in_specs=[
            pl.BlockSpec((NC, B, VSTEP), lambda i: (0, 0, jnp.minimum(i, NKM - 1))),
            pl.BlockSpec((VSTEP, D), lambda i: (jnp.minimum(i, NKM - 1), 0)),
            pl.BlockSpec((NC, B, V_TAIL), lambda i: (0, 0, 0)),
            pl.BlockSpec((V_TAIL, D), lambda i: (0, 0)),
            pl.BlockSpec((B, D), lambda i: (0, 0)),
            pl.BlockSpec(memory_space=pltpu.SMEM),
        ],
        out_specs=pl.BlockSpec((B, B), lambda i: (0, 0)),
        out_shape=jax.ShapeDtypeStruct((B, B), jnp.float32),
        scratch_shapes=[
            pltpu.VMEM((B, D), jnp.float32),
            pltpu.VMEM((B, D), jnp.float32),
        ],
    )(counts_bm, adapter_feats, ct, at, caption_embed, ls2d)


def kernel(adapter_feats, v2p_map, batch_idxs, caption_embed, caption_idx, logit_scale):
    del caption_idx
    counts_bm = _histogram(v2p_map, batch_idxs).reshape(NC, B, N_VOXELS)
    ct = lax.slice(counts_bm, (0, 0, NKM * VSTEP), (NC, B, N_VOXELS))
    at = lax.slice(adapter_feats, (NKM * VSTEP, 0), (N_VOXELS, D))
    ls2d = jnp.reshape(logit_scale, (1, 1))
    return _pool_logits(counts_bm, adapter_feats, ct, at, caption_embed, ls2d)
